# Initial kernel scaffold; baseline (speedup 1.0000x reference)
import jax
import jax.numpy as jnp
from jax.experimental import pallas as pl


def kernel(x, edge_index, batch, W_emb, b_emb, g_emb, be_emb, Wc0, bc0, gn0, bn0, Wc1, bc1, gn1, bn1, Wc2, bc2, gn2, bn2, W_r1, b_r1, W_r2, b_r2):
    raise NotImplementedError("write your pallas kernel here")


# speedup vs baseline: 13.7166x; 13.7166x over previous
import functools

import jax
import jax.numpy as jnp
from jax import lax
from jax.experimental import pallas as pl
from jax.experimental.pallas import tpu as pltpu
from jax.experimental.pallas import tpu_sc as plsc

_N = 10000
_NP = 10240
_E = 320000
_D = 128
_G = 64
_OUT = 12

_NC = 2
_NS = 16
_NW = _NC * _NS
_CK = 128
_NCHUNK = _E // _CK
_SC_ITERS = -(-_NCHUNK // _NW)
_RPS = _NP // _NS
_DW = 16

_B = 2000
_GRID = _N // _B

_HIGH = jax.lax.Precision.HIGHEST


def _mesh():
    return plsc.VectorSubcoreMesh(core_axis_name="c", subcore_axis_name="s")


@functools.partial(
    pl.kernel,
    mesh=_mesh(),
    out_type=[jax.ShapeDtypeStruct((_NP, _DW), jnp.float32)] * 2,
    scratch_types=[
        pltpu.VMEM((_CK,), jnp.int32),
        pltpu.VMEM((_CK, _DW), jnp.float32),
        pltpu.VMEM_SHARED((_NP, _DW), jnp.float32),
    ],
)
def _deg_pass(dst_hbm, out0_hbm, out1_hbm, didx, fill, acc):
    c = lax.axis_index("c")
    s = lax.axis_index("s")
    w = s * _NC + c

    def _fill(val):
        v = jnp.full((16,), val, jnp.float32)

        def body(i, carry):
            fill[i, pl.ds(0, 16)] = v
            return carry

        lax.fori_loop(0, _CK, body, 0)

    _fill(0.0)
    base_row = s * _RPS
    for j in range(_RPS // _CK):
        pltpu.sync_copy(fill, acc.at[pl.ds(base_row + j * _CK, _CK)])
    plsc.subcore_barrier()

    _fill(1.0)

    def body(i, carry):
        k = i * _NW + w

        @pl.when(k < _NCHUNK)
        def _():
            pltpu.sync_copy(dst_hbm.at[pl.ds(k * _CK, _CK)], didx)
            pltpu.sync_copy(fill, acc.at[didx], add=True)

        return carry

    lax.fori_loop(0, _SC_ITERS, body, 0)
    plsc.subcore_barrier()

    @pl.when(c == 0)
    def _():
        pltpu.sync_copy(acc.at[pl.ds(base_row, _RPS)],
                        out0_hbm.at[pl.ds(base_row, _RPS)])

    @pl.when(c == 1)
    def _():
        pltpu.sync_copy(acc.at[pl.ds(base_row, _RPS)],
                        out1_hbm.at[pl.ds(base_row, _RPS)])


@functools.partial(
    pl.kernel,
    mesh=_mesh(),
    out_type=[jax.ShapeDtypeStruct((_NP, _D), jnp.float32)] * 2,
    scratch_types=[
        pltpu.VMEM((_CK,), jnp.int32),
        pltpu.VMEM((_CK,), jnp.int32),
        pltpu.VMEM((_CK, _D), jnp.float32),
        pltpu.VMEM_SHARED((_NP, _D), jnp.float32),
        pltpu.SemaphoreType.DMA,
    ],
)
def _edge_pass(g_hbm, src_hbm, dst_hbm, out0_hbm, out1_hbm, sidx, didx, rows,
               acc, sem):
    c = lax.axis_index("c")
    s = lax.axis_index("s")
    w = s * _NC + c

    zero = jnp.zeros((16,), jnp.float32)

    def zbody(i, carry):
        for j in range(_D // 16):
            rows[i, pl.ds(j * 16, 16)] = zero
        return carry

    lax.fori_loop(0, _CK, zbody, 0)

    base_row = s * _RPS
    for j in range(_RPS // _CK):
        pltpu.sync_copy(rows, acc.at[pl.ds(base_row + j * _CK, _CK)])
    plsc.subcore_barrier()

    def body(i, carry):
        k = i * _NW + w

        @pl.when(k < _NCHUNK)
        def _():
            base = k * _CK
            pltpu.sync_copy(src_hbm.at[pl.ds(base, _CK)], sidx)
            pltpu.async_copy(g_hbm.at[sidx], rows, sem).wait()
            pltpu.sync_copy(dst_hbm.at[pl.ds(base, _CK)], didx)
            pltpu.sync_copy(rows, acc.at[didx], add=True)

        return carry

    lax.fori_loop(0, _SC_ITERS, body, 0)
    plsc.subcore_barrier()

    @pl.when(c == 0)
    def _():
        pltpu.sync_copy(acc.at[pl.ds(base_row, _RPS)],
                        out0_hbm.at[pl.ds(base_row, _RPS)])

    @pl.when(c == 1)
    def _():
        pltpu.sync_copy(acc.at[pl.ds(base_row, _RPS)],
                        out1_hbm.at[pl.ds(base_row, _RPS)])


def _layer_norm_block(t, g, b):
    mu = jnp.mean(t, axis=-1, keepdims=True)
    var = jnp.mean((t - mu) ** 2, axis=-1, keepdims=True)
    return (t - mu) * lax.rsqrt(var + 1e-5) * g + b


def _dinv_block(dega, degb):
    deg = dega[:, :1] + degb[:, :1] + 1.0
    return lax.rsqrt(deg)


def _emb_body(x_ref, we_ref, be_ref, ge_ref, bee_ref, dega_ref, degb_ref,
              wc_ref, h_ref, g_ref):
    t = jnp.dot(x_ref[...], we_ref[...], precision=_HIGH,
                preferred_element_type=jnp.float32) + be_ref[...]
    h = jnp.maximum(_layer_norm_block(t, ge_ref[...], bee_ref[...]), 0.0)
    h_ref[...] = h
    dinv = _dinv_block(dega_ref[...], degb_ref[...])
    g_ref[...] = dinv * jnp.dot(h, wc_ref[...], precision=_HIGH,
                                preferred_element_type=jnp.float32)


def _post_body(sa_ref, sb_ref, g_ref, hp_ref, dega_ref, degb_ref, b_ref,
               gn_ref, bn_ref, wc_ref, h_ref, gout_ref):
    dinv = _dinv_block(dega_ref[...], degb_ref[...])
    conv = dinv * (sa_ref[...] + sb_ref[...] + g_ref[...]) + b_ref[...]
    h = jnp.maximum(_layer_norm_block(conv, gn_ref[...], bn_ref[...]), 0.0)
    h = h + hp_ref[...]
    h_ref[...] = h
    gout_ref[...] = dinv * jnp.dot(h, wc_ref[...], precision=_HIGH,
                                   preferred_element_type=jnp.float32)


def _final_body(sa_ref, sb_ref, g_ref, hp_ref, dega_ref, degb_ref, b_ref,
                gn_ref, bn_ref, batch_ref, wr1_ref, br1_ref, wr2_ref, br2_ref,
                out_ref, sums_ref, cnt_ref):
    i = pl.program_id(0)
    dinv = _dinv_block(dega_ref[...], degb_ref[...])
    conv = dinv * (sa_ref[...] + sb_ref[...] + g_ref[...]) + b_ref[...]
    h = jnp.maximum(_layer_norm_block(conv, gn_ref[...], bn_ref[...]), 0.0)
    h = h + hp_ref[...]

    row = batch_ref[0]
    seg = lax.broadcasted_iota(jnp.int32, (_G, _B), 0)
    onehot = (row == seg).astype(jnp.float32)
    psum = jnp.dot(onehot, h, precision=_HIGH,
                   preferred_element_type=jnp.float32)
    pcnt = jnp.dot(onehot, jnp.ones((_B, 8), jnp.float32), precision=_HIGH,
                   preferred_element_type=jnp.float32)

    @pl.when(i == 0)
    def _():
        sums_ref[...] = psum
        cnt_ref[...] = pcnt

    @pl.when(i > 0)
    def _():
        sums_ref[...] += psum
        cnt_ref[...] += pcnt

    @pl.when(i == _GRID - 1)
    def _():
        pooled = sums_ref[...] / jnp.maximum(cnt_ref[:, :1], 1.0)
        r = jnp.maximum(jnp.dot(pooled, wr1_ref[...], precision=_HIGH,
                                preferred_element_type=jnp.float32)
                        + br1_ref[...], 0.0)
        out_ref[...] = jnp.dot(r, wr2_ref[...], precision=_HIGH,
                               preferred_element_type=jnp.float32) + br2_ref[...]


def _row_spec():
    return pl.BlockSpec((_B, _D), lambda i: (i, 0))


def _deg_spec():
    return pl.BlockSpec((_B, _DW), lambda i: (i, 0))


def _full_spec(shape):
    return pl.BlockSpec(shape, lambda i: (0,) * len(shape))


def _emb_call(x, W_emb, b_emb, g_emb, be_emb, dega, degb, Wc0):
    return pl.pallas_call(
        _emb_body,
        grid=(_GRID,),
        in_specs=[
            _row_spec(),
            _full_spec((_D, _D)),
            _full_spec((1, _D)),
            _full_spec((1, _D)),
            _full_spec((1, _D)),
            _deg_spec(),
            _deg_spec(),
            _full_spec((_D, _D)),
        ],
        out_specs=[_row_spec(), _row_spec()],
        out_shape=[jax.ShapeDtypeStruct((_N, _D), jnp.float32)] * 2,
    )(x, W_emb, b_emb.reshape(1, _D), g_emb.reshape(1, _D),
      be_emb.reshape(1, _D), dega, degb, Wc0)


def _post_call(Sa, Sb, g, h, dega, degb, b, gn, bn, Wc_next):
    return pl.pallas_call(
        _post_body,
        grid=(_GRID,),
        in_specs=[
            _row_spec(),
            _row_spec(),
            _row_spec(),
            _row_spec(),
            _deg_spec(),
            _deg_spec(),
            _full_spec((1, _D)),
            _full_spec((1, _D)),
            _full_spec((1, _D)),
            _full_spec((_D, _D)),
        ],
        out_specs=[_row_spec(), _row_spec()],
        out_shape=[jax.ShapeDtypeStruct((_N, _D), jnp.float32)] * 2,
    )(Sa, Sb, g, h, dega, degb, b.reshape(1, _D), gn.reshape(1, _D),
      bn.reshape(1, _D), Wc_next)


def _final_call(Sa, Sb, g, h, dega, degb, b, gn, bn, batch3, W_r1, b_r1,
                W_r2, b_r2):
    return pl.pallas_call(
        _final_body,
        grid=(_GRID,),
        in_specs=[
            _row_spec(),
            _row_spec(),
            _row_spec(),
            _row_spec(),
            _deg_spec(),
            _deg_spec(),
            _full_spec((1, _D)),
            _full_spec((1, _D)),
            _full_spec((1, _D)),
            pl.BlockSpec((1, 1, _B), lambda i: (i, 0, 0)),
            _full_spec((_D, _G)),
            _full_spec((1, _G)),
            _full_spec((_G, _OUT)),
            _full_spec((1, _OUT)),
        ],
        out_specs=pl.BlockSpec((_G, _OUT), lambda i: (0, 0)),
        out_shape=jax.ShapeDtypeStruct((_G, _OUT), jnp.float32),
        scratch_shapes=[
            pltpu.VMEM((_G, _D), jnp.float32),
            pltpu.VMEM((_G, 8), jnp.float32),
        ],
    )(Sa, Sb, g, h, dega, degb, b.reshape(1, _D), gn.reshape(1, _D),
      bn.reshape(1, _D), batch3, W_r1, b_r1.reshape(1, _G),
      W_r2, b_r2.reshape(1, _OUT))


def kernel(x, edge_index, batch, W_emb, b_emb, g_emb, be_emb, Wc0, bc0, gn0,
           bn0, Wc1, bc1, gn1, bn1, Wc2, bc2, gn2, bn2, W_r1, b_r1, W_r2,
           b_r2):
    src = edge_index[0]
    dst = edge_index[1]
    dega, degb = _deg_pass(dst)

    h0, g0 = _emb_call(x, W_emb, b_emb, g_emb, be_emb, dega, degb, Wc0)
    S0a, S0b = _edge_pass(g0, src, dst)
    h1, g1 = _post_call(S0a, S0b, g0, h0, dega, degb, bc0, gn0, bn0, Wc1)
    S1a, S1b = _edge_pass(g1, src, dst)
    h2, g2 = _post_call(S1a, S1b, g1, h1, dega, degb, bc1, gn1, bn1, Wc2)
    S2a, S2b = _edge_pass(g2, src, dst)

    batch3 = batch.reshape(_GRID, 1, _B)
    return _final_call(S2a, S2b, g2, h2, dega, degb, bc2, gn2, bn2, batch3,
                       W_r1, b_r1, W_r2, b_r2)


# speedup vs baseline: 27.5459x; 2.0082x over previous
import functools

import jax
import jax.numpy as jnp
from jax import lax
from jax.experimental import pallas as pl
from jax.experimental.pallas import tpu as pltpu
from jax.experimental.pallas import tpu_sc as plsc

_N = 10000
_NP = 10240
_E = 320000
_D = 128
_G = 64
_OUT = 12

_NC = 2
_NS = 16
_NW = _NC * _NS
_CK = 128
_NCHUNK = _E // _CK
_CPW = 80
_CPW_LAST = _NCHUNK - _CPW * (_NW - 1)
_RPS = _NP // _NS
_DW = 16

_B = 2000
_GRID = _N // _B

_HIGH = jax.lax.Precision.HIGHEST


def _mesh():
    return plsc.VectorSubcoreMesh(core_axis_name="c", subcore_axis_name="s")


def _load_my_chunks(src2_hbm, buf, w):
    @pl.when(w < _NW - 1)
    def _():
        pltpu.sync_copy(src2_hbm.at[pl.ds(w * _CPW, _CPW)], buf)

    @pl.when(w == _NW - 1)
    def _():
        pltpu.sync_copy(src2_hbm.at[pl.ds((_NW - 1) * _CPW, _CPW_LAST)],
                        buf.at[pl.ds(0, _CPW_LAST)])


def _my_nchunks(w):
    return jnp.where(w < _NW - 1, _CPW, _CPW_LAST)


@functools.partial(
    pl.kernel,
    mesh=_mesh(),
    out_type=[jax.ShapeDtypeStruct((_NP, _DW), jnp.float32)] * 2,
    scratch_types=[
        pltpu.VMEM((_CPW, _CK), jnp.int32),
        pltpu.VMEM((_CK, _DW), jnp.float32),
        pltpu.VMEM_SHARED((_NP, _DW), jnp.float32),
    ],
)
def _deg_pass(dst2_hbm, out0_hbm, out1_hbm, didx_all, fill, acc):
    c = lax.axis_index("c")
    s = lax.axis_index("s")
    w = s * _NC + c

    def _fill(val):
        v = jnp.full((16,), val, jnp.float32)

        def body(i, carry):
            fill[i, pl.ds(0, 16)] = v
            return carry

        lax.fori_loop(0, _CK, body, 0)

    _fill(0.0)
    base_row = s * _RPS
    for j in range(_RPS // _CK):
        pltpu.sync_copy(fill, acc.at[pl.ds(base_row + j * _CK, _CK)])
    _load_my_chunks(dst2_hbm, didx_all, w)
    nchunks = _my_nchunks(w)
    plsc.subcore_barrier()

    _fill(1.0)

    def body(j, carry):
        @pl.when(j < nchunks)
        def _():
            pltpu.sync_copy(fill, acc.at[didx_all.at[j]], add=True)

        return carry

    lax.fori_loop(0, _CPW, body, 0)
    plsc.subcore_barrier()

    @pl.when(c == 0)
    def _():
        pltpu.sync_copy(acc.at[pl.ds(base_row, _RPS)],
                        out0_hbm.at[pl.ds(base_row, _RPS)])

    @pl.when(c == 1)
    def _():
        pltpu.sync_copy(acc.at[pl.ds(base_row, _RPS)],
                        out1_hbm.at[pl.ds(base_row, _RPS)])


@functools.partial(
    pl.kernel,
    mesh=_mesh(),
    out_type=[jax.ShapeDtypeStruct((_NP, _D), jnp.float32)] * 2,
    scratch_types=[
        pltpu.VMEM((_CPW, _CK), jnp.int32),
        pltpu.VMEM((4, _CK), jnp.int32),
        pltpu.VMEM((_CK, _D), jnp.float32),
        pltpu.VMEM((_CK, _D), jnp.float32),
        pltpu.VMEM_SHARED((_NP, _D), jnp.float32),
        pltpu.SemaphoreType.DMA,
        pltpu.SemaphoreType.DMA,
        pltpu.SemaphoreType.DMA,
        pltpu.SemaphoreType.DMA,
        pltpu.SemaphoreType.DMA,
        pltpu.SemaphoreType.DMA,
    ],
)
def _edge_pass(g_hbm, src2_hbm, dst2_hbm, out0_hbm, out1_hbm, didx_all, sidx,
               rows0, rows1, acc, gsem0, gsem1, isem0, isem1, isem2, isem3):
    c = lax.axis_index("c")
    s = lax.axis_index("s")
    w = s * _NC + c
    rows = (rows0, rows1)
    gsem = (gsem0, gsem1)
    isem = (isem0, isem1, isem2, isem3)
    start_chunk = w * _CPW
    nchunks = _my_nchunks(w)

    zero = jnp.zeros((16,), jnp.float32)

    def zbody(i, carry):
        for j in range(_D // 16):
            rows0[i, pl.ds(j * 16, 16)] = zero
        return carry

    lax.fori_loop(0, _CK, zbody, 0)

    base_row = s * _RPS
    for j in range(_RPS // _CK):
        pltpu.sync_copy(rows0, acc.at[pl.ds(base_row + j * _CK, _CK)])
    _load_my_chunks(dst2_hbm, didx_all, w)
    plsc.subcore_barrier()

    def _idx_load(j, slot):
        return pltpu.make_async_copy(src2_hbm.at[pl.ds(start_chunk + j, 1)],
                                     sidx.at[pl.ds(slot, 1)], isem[slot])

    def _gather(j, b, slot):
        return pltpu.make_async_copy(g_hbm.at[sidx.at[slot]], rows[b],
                                     gsem[b])

    for k in range(4):
        @pl.when(jnp.int32(k) < nchunks)
        def _(k=k):
            _idx_load(k, k).start()

    for k in range(2):
        @pl.when(jnp.int32(k) < nchunks)
        def _(k=k):
            _idx_load(k, k).wait()
            _gather(k, k, k).start()

    def body(i, carry):
        for k in range(4):
            j = i * 4 + k
            b = k % 2

            @pl.when(j < nchunks)
            def _(j=j, b=b, k=k):
                _gather(j, b, k).wait()
                pltpu.sync_copy(rows[b], acc.at[didx_all.at[j]], add=True)

                @pl.when(j + 4 < nchunks)
                def _():
                    _idx_load(j + 4, k).start()

                @pl.when(j + 2 < nchunks)
                def _():
                    slot2 = (k + 2) % 4
                    _idx_load(j + 2, slot2).wait()
                    _gather(j + 2, b, slot2).start()

        return carry

    lax.fori_loop(0, _CPW // 4, body, 0)
    plsc.subcore_barrier()

    @pl.when(c == 0)
    def _():
        pltpu.sync_copy(acc.at[pl.ds(base_row, _RPS)],
                        out0_hbm.at[pl.ds(base_row, _RPS)])

    @pl.when(c == 1)
    def _():
        pltpu.sync_copy(acc.at[pl.ds(base_row, _RPS)],
                        out1_hbm.at[pl.ds(base_row, _RPS)])


def _layer_norm_block(t, g, b):
    mu = jnp.mean(t, axis=-1, keepdims=True)
    var = jnp.mean((t - mu) ** 2, axis=-1, keepdims=True)
    return (t - mu) * lax.rsqrt(var + 1e-5) * g + b


def _dinv_block(dega, degb):
    deg = dega[:, :1] + degb[:, :1] + 1.0
    return lax.rsqrt(deg)


def _emb_body(x_ref, we_ref, be_ref, ge_ref, bee_ref, dega_ref, degb_ref,
              wc_ref, h_ref, g_ref):
    t = jnp.dot(x_ref[...], we_ref[...], precision=_HIGH,
                preferred_element_type=jnp.float32) + be_ref[...]
    h = jnp.maximum(_layer_norm_block(t, ge_ref[...], bee_ref[...]), 0.0)
    h_ref[...] = h
    dinv = _dinv_block(dega_ref[...], degb_ref[...])
    g_ref[...] = dinv * jnp.dot(h, wc_ref[...], precision=_HIGH,
                                preferred_element_type=jnp.float32)


def _post_body(sa_ref, sb_ref, g_ref, hp_ref, dega_ref, degb_ref, b_ref,
               gn_ref, bn_ref, wc_ref, h_ref, gout_ref):
    dinv = _dinv_block(dega_ref[...], degb_ref[...])
    conv = dinv * (sa_ref[...] + sb_ref[...] + g_ref[...]) + b_ref[...]
    h = jnp.maximum(_layer_norm_block(conv, gn_ref[...], bn_ref[...]), 0.0)
    h = h + hp_ref[...]
    h_ref[...] = h
    gout_ref[...] = dinv * jnp.dot(h, wc_ref[...], precision=_HIGH,
                                   preferred_element_type=jnp.float32)


def _final_body(sa_ref, sb_ref, g_ref, hp_ref, dega_ref, degb_ref, b_ref,
                gn_ref, bn_ref, batch_ref, wr1_ref, br1_ref, wr2_ref, br2_ref,
                out_ref, sums_ref, cnt_ref):
    i = pl.program_id(0)
    dinv = _dinv_block(dega_ref[...], degb_ref[...])
    conv = dinv * (sa_ref[...] + sb_ref[...] + g_ref[...]) + b_ref[...]
    h = jnp.maximum(_layer_norm_block(conv, gn_ref[...], bn_ref[...]), 0.0)
    h = h + hp_ref[...]

    row = batch_ref[0]
    seg = lax.broadcasted_iota(jnp.int32, (_G, _B), 0)
    onehot = (row == seg).astype(jnp.float32)
    psum = jnp.dot(onehot, h, precision=_HIGH,
                   preferred_element_type=jnp.float32)
    pcnt = jnp.dot(onehot, jnp.ones((_B, 8), jnp.float32), precision=_HIGH,
                   preferred_element_type=jnp.float32)

    @pl.when(i == 0)
    def _():
        sums_ref[...] = psum
        cnt_ref[...] = pcnt

    @pl.when(i > 0)
    def _():
        sums_ref[...] += psum
        cnt_ref[...] += pcnt

    @pl.when(i == _GRID - 1)
    def _():
        pooled = sums_ref[...] / jnp.maximum(cnt_ref[:, :1], 1.0)
        r = jnp.maximum(jnp.dot(pooled, wr1_ref[...], precision=_HIGH,
                                preferred_element_type=jnp.float32)
                        + br1_ref[...], 0.0)
        out_ref[...] = jnp.dot(r, wr2_ref[...], precision=_HIGH,
                               preferred_element_type=jnp.float32) + br2_ref[...]


def _row_spec():
    return pl.BlockSpec((_B, _D), lambda i: (i, 0))


def _deg_spec():
    return pl.BlockSpec((_B, _DW), lambda i: (i, 0))


def _full_spec(shape):
    return pl.BlockSpec(shape, lambda i: (0,) * len(shape))


def _emb_call(x, W_emb, b_emb, g_emb, be_emb, dega, degb, Wc0):
    return pl.pallas_call(
        _emb_body,
        grid=(_GRID,),
        in_specs=[
            _row_spec(),
            _full_spec((_D, _D)),
            _full_spec((1, _D)),
            _full_spec((1, _D)),
            _full_spec((1, _D)),
            _deg_spec(),
            _deg_spec(),
            _full_spec((_D, _D)),
        ],
        out_specs=[_row_spec(), _row_spec()],
        out_shape=[jax.ShapeDtypeStruct((_N, _D), jnp.float32)] * 2,
    )(x, W_emb, b_emb.reshape(1, _D), g_emb.reshape(1, _D),
      be_emb.reshape(1, _D), dega, degb, Wc0)


def _post_call(Sa, Sb, g, h, dega, degb, b, gn, bn, Wc_next):
    return pl.pallas_call(
        _post_body,
        grid=(_GRID,),
        in_specs=[
            _row_spec(),
            _row_spec(),
            _row_spec(),
            _row_spec(),
            _deg_spec(),
            _deg_spec(),
            _full_spec((1, _D)),
            _full_spec((1, _D)),
            _full_spec((1, _D)),
            _full_spec((_D, _D)),
        ],
        out_specs=[_row_spec(), _row_spec()],
        out_shape=[jax.ShapeDtypeStruct((_N, _D), jnp.float32)] * 2,
    )(Sa, Sb, g, h, dega, degb, b.reshape(1, _D), gn.reshape(1, _D),
      bn.reshape(1, _D), Wc_next)


def _final_call(Sa, Sb, g, h, dega, degb, b, gn, bn, batch3, W_r1, b_r1,
                W_r2, b_r2):
    return pl.pallas_call(
        _final_body,
        grid=(_GRID,),
        in_specs=[
            _row_spec(),
            _row_spec(),
            _row_spec(),
            _row_spec(),
            _deg_spec(),
            _deg_spec(),
            _full_spec((1, _D)),
            _full_spec((1, _D)),
            _full_spec((1, _D)),
            pl.BlockSpec((1, 1, _B), lambda i: (i, 0, 0)),
            _full_spec((_D, _G)),
            _full_spec((1, _G)),
            _full_spec((_G, _OUT)),
            _full_spec((1, _OUT)),
        ],
        out_specs=pl.BlockSpec((_G, _OUT), lambda i: (0, 0)),
        out_shape=jax.ShapeDtypeStruct((_G, _OUT), jnp.float32),
        scratch_shapes=[
            pltpu.VMEM((_G, _D), jnp.float32),
            pltpu.VMEM((_G, 8), jnp.float32),
        ],
    )(Sa, Sb, g, h, dega, degb, b.reshape(1, _D), gn.reshape(1, _D),
      bn.reshape(1, _D), batch3, W_r1, b_r1.reshape(1, _G),
      W_r2, b_r2.reshape(1, _OUT))


def kernel(x, edge_index, batch, W_emb, b_emb, g_emb, be_emb, Wc0, bc0, gn0,
           bn0, Wc1, bc1, gn1, bn1, Wc2, bc2, gn2, bn2, W_r1, b_r1, W_r2,
           b_r2):
    src2 = edge_index[0].reshape(_NCHUNK, _CK)
    dst2 = edge_index[1].reshape(_NCHUNK, _CK)
    dega, degb = _deg_pass(dst2)

    h0, g0 = _emb_call(x, W_emb, b_emb, g_emb, be_emb, dega, degb, Wc0)
    S0a, S0b = _edge_pass(g0, src2, dst2)
    h1, g1 = _post_call(S0a, S0b, g0, h0, dega, degb, bc0, gn0, bn0, Wc1)
    S1a, S1b = _edge_pass(g1, src2, dst2)
    h2, g2 = _post_call(S1a, S1b, g1, h1, dega, degb, bc1, gn1, bn1, Wc2)
    S2a, S2b = _edge_pass(g2, src2, dst2)

    batch3 = batch.reshape(_GRID, 1, _B)
    return _final_call(S2a, S2b, g2, h2, dega, degb, bc2, gn2, bn2, batch3,
                       W_r1, b_r1, W_r2, b_r2)


# speedup vs baseline: 27.5894x; 1.0016x over previous
import functools

import jax
import jax.numpy as jnp
from jax import lax
from jax.experimental import pallas as pl
from jax.experimental.pallas import tpu as pltpu
from jax.experimental.pallas import tpu_sc as plsc

_N = 10000
_NP = 10240
_E = 320000
_D = 128
_G = 64
_OUT = 12

_NC = 2
_NS = 16
_NW = _NC * _NS
_CK = 128
_NCHUNK = _E // _CK
_CPW = 80
_CPW_LAST = _NCHUNK - _CPW * (_NW - 1)
_RPS = _NP // _NS
_DW = 16

_B = 2000
_GRID = _N // _B

_HIGH = jax.lax.Precision.HIGHEST


def _mesh():
    return plsc.VectorSubcoreMesh(core_axis_name="c", subcore_axis_name="s")


def _load_my_chunks(src2_hbm, buf, w):
    @pl.when(w < _NW - 1)
    def _():
        pltpu.sync_copy(src2_hbm.at[pl.ds(w * _CPW, _CPW)], buf)

    @pl.when(w == _NW - 1)
    def _():
        pltpu.sync_copy(src2_hbm.at[pl.ds((_NW - 1) * _CPW, _CPW_LAST)],
                        buf.at[pl.ds(0, _CPW_LAST)])


def _my_nchunks(w):
    return jnp.where(w < _NW - 1, _CPW, _CPW_LAST)


@functools.partial(
    pl.kernel,
    mesh=_mesh(),
    out_type=[jax.ShapeDtypeStruct((_NP, _DW), jnp.float32)] * 2,
    scratch_types=[
        pltpu.VMEM((_CPW, _CK), jnp.int32),
        pltpu.VMEM((_CK, _DW), jnp.float32),
        pltpu.VMEM_SHARED((_NP, _DW), jnp.float32),
    ],
)
def _deg_pass(dst2_hbm, out0_hbm, out1_hbm, didx_all, fill, acc):
    c = lax.axis_index("c")
    s = lax.axis_index("s")
    w = s * _NC + c

    def _fill(val):
        v = jnp.full((16,), val, jnp.float32)

        def body(i, carry):
            fill[i, pl.ds(0, 16)] = v
            return carry

        lax.fori_loop(0, _CK, body, 0)

    _fill(0.0)
    base_row = s * _RPS
    for j in range(_RPS // _CK):
        pltpu.sync_copy(fill, acc.at[pl.ds(base_row + j * _CK, _CK)])
    _load_my_chunks(dst2_hbm, didx_all, w)
    nchunks = _my_nchunks(w)
    plsc.subcore_barrier()

    _fill(1.0)

    def body(j, carry):
        @pl.when(j < nchunks)
        def _():
            pltpu.sync_copy(fill, acc.at[didx_all.at[j]], add=True)

        return carry

    lax.fori_loop(0, _CPW, body, 0)
    plsc.subcore_barrier()

    @pl.when(c == 0)
    def _():
        pltpu.sync_copy(acc.at[pl.ds(base_row, _RPS)],
                        out0_hbm.at[pl.ds(base_row, _RPS)])

    @pl.when(c == 1)
    def _():
        pltpu.sync_copy(acc.at[pl.ds(base_row, _RPS)],
                        out1_hbm.at[pl.ds(base_row, _RPS)])


@functools.partial(
    pl.kernel,
    mesh=_mesh(),
    out_type=[jax.ShapeDtypeStruct((_NP, _D), jnp.float32)] * 2,
    scratch_types=[
        pltpu.VMEM((4, 2, _CK // 2), jnp.int32),
        pltpu.VMEM((4, _CK), jnp.int32),
        pltpu.VMEM((_CK, _D), jnp.float32),
        pltpu.VMEM((_CK, _D), jnp.float32),
        pltpu.VMEM_SHARED((_NP, _D), jnp.float32),
        pltpu.SemaphoreType.DMA,
        pltpu.SemaphoreType.DMA,
        pltpu.SemaphoreType.DMA,
        pltpu.SemaphoreType.DMA,
        pltpu.SemaphoreType.DMA,
        pltpu.SemaphoreType.DMA,
        pltpu.SemaphoreType.DMA,
        pltpu.SemaphoreType.DMA,
        pltpu.SemaphoreType.DMA,
        pltpu.SemaphoreType.DMA,
        pltpu.SemaphoreType.DMA,
        pltpu.SemaphoreType.DMA,
    ],
)
def _edge_pass(g_hbm, src2_hbm, dsth_hbm, out0_hbm, out1_hbm, didx, sidx,
               rows0, rows1, acc, gsem0, gsem1, ssem0, ssem1, isem0, isem1,
               isem2, isem3, dsem0, dsem1, dsem2, dsem3):
    c = lax.axis_index("c")
    s = lax.axis_index("s")
    w = s * _NC + c
    rows = (rows0, rows1)
    gsem = (gsem0, gsem1)
    ssem = (ssem0, ssem1)
    isem = (isem0, isem1, isem2, isem3)
    dsem = (dsem0, dsem1, dsem2, dsem3)
    start_chunk = w * _CPW
    nchunks = _my_nchunks(w)

    zero = jnp.zeros((16,), jnp.float32)

    def zbody(i, carry):
        for j in range(_D // 16):
            rows0[i, pl.ds(j * 16, 16)] = zero
        return carry

    lax.fori_loop(0, _CK, zbody, 0)

    base_row = s * _RPS
    for j in range(_RPS // _CK):
        pltpu.sync_copy(rows0, acc.at[pl.ds(base_row + j * _CK, _CK)])
    plsc.subcore_barrier()

    def _idx_load(j, slot):
        return pltpu.make_async_copy(src2_hbm.at[pl.ds(start_chunk + j, 1)],
                                     sidx.at[pl.ds(slot, 1)], isem[slot])

    def _didx_load(j, slot):
        return pltpu.make_async_copy(
            dsth_hbm.at[pl.ds(start_chunk + j, 1)],
            didx.at[pl.ds(slot, 1)], dsem[slot])

    def _gather(j, b, slot):
        return pltpu.make_async_copy(g_hbm.at[sidx.at[slot]], rows[b],
                                     gsem[b])

    def _scatter_half(b, slot, h):
        return pltpu.make_async_copy(
            rows[b].at[pl.ds(h * (_CK // 2), _CK // 2)],
            acc.at[didx.at[slot, h]], ssem[b])

    for k in range(4):
        @pl.when(jnp.int32(k) < nchunks)
        def _(k=k):
            _idx_load(k, k).start()
            _didx_load(k, k).start()

    for k in range(2):
        @pl.when(jnp.int32(k) < nchunks)
        def _(k=k):
            _idx_load(k, k).wait()
            _gather(k, k, k).start()

    def body(i, carry):
        for k in range(4):
            j = i * 4 + k
            b = k % 2

            @pl.when(j < nchunks)
            def _(j=j, b=b, k=k):
                _gather(j, b, k).wait()
                _didx_load(j, k).wait()
                _scatter_half(b, k, 0).start(add=True)
                _scatter_half(b, k, 1).start(add=True)
                _scatter_half(b, k, 0).wait()
                _scatter_half(b, k, 1).wait()

                @pl.when(j + 4 < nchunks)
                def _():
                    _idx_load(j + 4, k).start()
                    _didx_load(j + 4, k).start()

                @pl.when(j + 2 < nchunks)
                def _():
                    slot2 = (k + 2) % 4
                    _idx_load(j + 2, slot2).wait()
                    _gather(j + 2, b, slot2).start()

        return carry

    lax.fori_loop(0, _CPW // 4, body, 0)
    plsc.subcore_barrier()

    @pl.when(c == 0)
    def _():
        pltpu.sync_copy(acc.at[pl.ds(base_row, _RPS)],
                        out0_hbm.at[pl.ds(base_row, _RPS)])

    @pl.when(c == 1)
    def _():
        pltpu.sync_copy(acc.at[pl.ds(base_row, _RPS)],
                        out1_hbm.at[pl.ds(base_row, _RPS)])


def _layer_norm_block(t, g, b):
    mu = jnp.mean(t, axis=-1, keepdims=True)
    var = jnp.mean((t - mu) ** 2, axis=-1, keepdims=True)
    return (t - mu) * lax.rsqrt(var + 1e-5) * g + b


def _dinv_block(dega, degb):
    deg = dega[:, :1] + degb[:, :1] + 1.0
    return lax.rsqrt(deg)


def _emb_body(x_ref, we_ref, be_ref, ge_ref, bee_ref, dega_ref, degb_ref,
              wc_ref, h_ref, g_ref):
    t = jnp.dot(x_ref[...], we_ref[...], precision=_HIGH,
                preferred_element_type=jnp.float32) + be_ref[...]
    h = jnp.maximum(_layer_norm_block(t, ge_ref[...], bee_ref[...]), 0.0)
    h_ref[...] = h
    dinv = _dinv_block(dega_ref[...], degb_ref[...])
    g_ref[...] = dinv * jnp.dot(h, wc_ref[...], precision=_HIGH,
                                preferred_element_type=jnp.float32)


def _post_body(sa_ref, sb_ref, g_ref, hp_ref, dega_ref, degb_ref, b_ref,
               gn_ref, bn_ref, wc_ref, h_ref, gout_ref):
    dinv = _dinv_block(dega_ref[...], degb_ref[...])
    conv = dinv * (sa_ref[...] + sb_ref[...] + g_ref[...]) + b_ref[...]
    h = jnp.maximum(_layer_norm_block(conv, gn_ref[...], bn_ref[...]), 0.0)
    h = h + hp_ref[...]
    h_ref[...] = h
    gout_ref[...] = dinv * jnp.dot(h, wc_ref[...], precision=_HIGH,
                                   preferred_element_type=jnp.float32)


def _final_body(sa_ref, sb_ref, g_ref, hp_ref, dega_ref, degb_ref, b_ref,
                gn_ref, bn_ref, batch_ref, wr1_ref, br1_ref, wr2_ref, br2_ref,
                out_ref, sums_ref, cnt_ref):
    i = pl.program_id(0)
    dinv = _dinv_block(dega_ref[...], degb_ref[...])
    conv = dinv * (sa_ref[...] + sb_ref[...] + g_ref[...]) + b_ref[...]
    h = jnp.maximum(_layer_norm_block(conv, gn_ref[...], bn_ref[...]), 0.0)
    h = h + hp_ref[...]

    row = batch_ref[0]
    seg = lax.broadcasted_iota(jnp.int32, (_G, _B), 0)
    onehot = (row == seg).astype(jnp.float32)
    psum = jnp.dot(onehot, h, precision=_HIGH,
                   preferred_element_type=jnp.float32)
    pcnt = jnp.dot(onehot, jnp.ones((_B, 8), jnp.float32), precision=_HIGH,
                   preferred_element_type=jnp.float32)

    @pl.when(i == 0)
    def _():
        sums_ref[...] = psum
        cnt_ref[...] = pcnt

    @pl.when(i > 0)
    def _():
        sums_ref[...] += psum
        cnt_ref[...] += pcnt

    @pl.when(i == _GRID - 1)
    def _():
        pooled = sums_ref[...] / jnp.maximum(cnt_ref[:, :1], 1.0)
        r = jnp.maximum(jnp.dot(pooled, wr1_ref[...], precision=_HIGH,
                                preferred_element_type=jnp.float32)
                        + br1_ref[...], 0.0)
        out_ref[...] = jnp.dot(r, wr2_ref[...], precision=_HIGH,
                               preferred_element_type=jnp.float32) + br2_ref[...]


def _row_spec():
    return pl.BlockSpec((_B, _D), lambda i: (i, 0))


def _deg_spec():
    return pl.BlockSpec((_B, _DW), lambda i: (i, 0))


def _full_spec(shape):
    return pl.BlockSpec(shape, lambda i: (0,) * len(shape))


def _emb_call(x, W_emb, b_emb, g_emb, be_emb, dega, degb, Wc0):
    return pl.pallas_call(
        _emb_body,
        grid=(_GRID,),
        in_specs=[
            _row_spec(),
            _full_spec((_D, _D)),
            _full_spec((1, _D)),
            _full_spec((1, _D)),
            _full_spec((1, _D)),
            _deg_spec(),
            _deg_spec(),
            _full_spec((_D, _D)),
        ],
        out_specs=[_row_spec(), _row_spec()],
        out_shape=[jax.ShapeDtypeStruct((_N, _D), jnp.float32)] * 2,
    )(x, W_emb, b_emb.reshape(1, _D), g_emb.reshape(1, _D),
      be_emb.reshape(1, _D), dega, degb, Wc0)


def _post_call(Sa, Sb, g, h, dega, degb, b, gn, bn, Wc_next):
    return pl.pallas_call(
        _post_body,
        grid=(_GRID,),
        in_specs=[
            _row_spec(),
            _row_spec(),
            _row_spec(),
            _row_spec(),
            _deg_spec(),
            _deg_spec(),
            _full_spec((1, _D)),
            _full_spec((1, _D)),
            _full_spec((1, _D)),
            _full_spec((_D, _D)),
        ],
        out_specs=[_row_spec(), _row_spec()],
        out_shape=[jax.ShapeDtypeStruct((_N, _D), jnp.float32)] * 2,
    )(Sa, Sb, g, h, dega, degb, b.reshape(1, _D), gn.reshape(1, _D),
      bn.reshape(1, _D), Wc_next)


def _final_call(Sa, Sb, g, h, dega, degb, b, gn, bn, batch3, W_r1, b_r1,
                W_r2, b_r2):
    return pl.pallas_call(
        _final_body,
        grid=(_GRID,),
        in_specs=[
            _row_spec(),
            _row_spec(),
            _row_spec(),
            _row_spec(),
            _deg_spec(),
            _deg_spec(),
            _full_spec((1, _D)),
            _full_spec((1, _D)),
            _full_spec((1, _D)),
            pl.BlockSpec((1, 1, _B), lambda i: (i, 0, 0)),
            _full_spec((_D, _G)),
            _full_spec((1, _G)),
            _full_spec((_G, _OUT)),
            _full_spec((1, _OUT)),
        ],
        out_specs=pl.BlockSpec((_G, _OUT), lambda i: (0, 0)),
        out_shape=jax.ShapeDtypeStruct((_G, _OUT), jnp.float32),
        scratch_shapes=[
            pltpu.VMEM((_G, _D), jnp.float32),
            pltpu.VMEM((_G, 8), jnp.float32),
        ],
    )(Sa, Sb, g, h, dega, degb, b.reshape(1, _D), gn.reshape(1, _D),
      bn.reshape(1, _D), batch3, W_r1, b_r1.reshape(1, _G),
      W_r2, b_r2.reshape(1, _OUT))


def kernel(x, edge_index, batch, W_emb, b_emb, g_emb, be_emb, Wc0, bc0, gn0,
           bn0, Wc1, bc1, gn1, bn1, Wc2, bc2, gn2, bn2, W_r1, b_r1, W_r2,
           b_r2):
    src2 = edge_index[0].reshape(_NCHUNK, _CK)
    dst2 = edge_index[1].reshape(_NCHUNK, _CK)
    dsth = edge_index[1].reshape(_NCHUNK, 2, _CK // 2)
    dega, degb = _deg_pass(dst2)

    h0, g0 = _emb_call(x, W_emb, b_emb, g_emb, be_emb, dega, degb, Wc0)
    S0a, S0b = _edge_pass(g0, src2, dsth)
    h1, g1 = _post_call(S0a, S0b, g0, h0, dega, degb, bc0, gn0, bn0, Wc1)
    S1a, S1b = _edge_pass(g1, src2, dsth)
    h2, g2 = _post_call(S1a, S1b, g1, h1, dega, degb, bc1, gn1, bn1, Wc2)
    S2a, S2b = _edge_pass(g2, src2, dsth)

    batch3 = batch.reshape(_GRID, 1, _B)
    return _final_call(S2a, S2b, g2, h2, dega, degb, bc2, gn2, bn2, batch3,
                       W_r1, b_r1, W_r2, b_r2)


# speedup vs baseline: 28.8460x; 1.0455x over previous
import functools

import jax
import jax.numpy as jnp
from jax import lax
from jax.experimental import pallas as pl
from jax.experimental.pallas import tpu as pltpu
from jax.experimental.pallas import tpu_sc as plsc

_N = 10000
_NP = 10240
_E = 320000
_D = 128
_G = 64
_OUT = 12

_NC = 2
_NS = 16
_NW = _NC * _NS
_CK = 128
_NCHUNK = _E // _CK
_CPW = 80
_CPW_LAST = _NCHUNK - _CPW * (_NW - 1)
_RPS = _NP // _NS
_DW = 16

_B = 2000
_GRID = _N // _B

_HIGH = jax.lax.Precision.HIGHEST


def _mesh():
    return plsc.VectorSubcoreMesh(core_axis_name="c", subcore_axis_name="s")


def _load_my_chunks(src2_hbm, buf, w):
    @pl.when(w < _NW - 1)
    def _():
        pltpu.sync_copy(src2_hbm.at[pl.ds(w * _CPW, _CPW)], buf)

    @pl.when(w == _NW - 1)
    def _():
        pltpu.sync_copy(src2_hbm.at[pl.ds((_NW - 1) * _CPW, _CPW_LAST)],
                        buf.at[pl.ds(0, _CPW_LAST)])


def _my_nchunks(w):
    return jnp.where(w < _NW - 1, _CPW, _CPW_LAST)


@functools.partial(
    pl.kernel,
    mesh=_mesh(),
    out_type=[jax.ShapeDtypeStruct((_NP, _DW), jnp.float32)] * 2,
    scratch_types=[
        pltpu.VMEM((_CPW, _CK), jnp.int32),
        pltpu.VMEM((_CK, _DW), jnp.float32),
        pltpu.VMEM_SHARED((_NP, _DW), jnp.float32),
    ],
)
def _deg_pass(edge3_hbm, out0_hbm, out1_hbm, didx_all, fill, acc):
    c = lax.axis_index("c")
    s = lax.axis_index("s")
    w = s * _NC + c

    def _fill(val):
        v = jnp.full((16,), val, jnp.float32)

        def body(i, carry):
            fill[i, pl.ds(0, 16)] = v
            return carry

        lax.fori_loop(0, _CK, body, 0)

    _fill(0.0)
    base_row = s * _RPS
    for j in range(_RPS // _CK):
        pltpu.sync_copy(fill, acc.at[pl.ds(base_row + j * _CK, _CK)])
    @pl.when(w < _NW - 1)
    def _():
        pltpu.sync_copy(edge3_hbm.at[1, pl.ds(w * _CPW, _CPW)], didx_all)

    @pl.when(w == _NW - 1)
    def _():
        pltpu.sync_copy(edge3_hbm.at[1, pl.ds((_NW - 1) * _CPW, _CPW_LAST)],
                        didx_all.at[pl.ds(0, _CPW_LAST)])

    nchunks = _my_nchunks(w)
    plsc.subcore_barrier()

    _fill(1.0)

    def body(j, carry):
        @pl.when(j < nchunks)
        def _():
            pltpu.sync_copy(fill, acc.at[didx_all.at[j]], add=True)

        return carry

    lax.fori_loop(0, _CPW, body, 0)
    plsc.subcore_barrier()

    @pl.when(c == 0)
    def _():
        pltpu.sync_copy(acc.at[pl.ds(base_row, _RPS)],
                        out0_hbm.at[pl.ds(base_row, _RPS)])

    @pl.when(c == 1)
    def _():
        pltpu.sync_copy(acc.at[pl.ds(base_row, _RPS)],
                        out1_hbm.at[pl.ds(base_row, _RPS)])


@functools.partial(
    pl.kernel,
    mesh=_mesh(),
    out_type=[jax.ShapeDtypeStruct((_NP, _D), jnp.float32)] * 2,
    scratch_types=[
        pltpu.VMEM((4, 2, _CK // 2), jnp.int32),
        pltpu.VMEM((4, _CK), jnp.int32),
        pltpu.VMEM((_CK, _D), jnp.float32),
        pltpu.VMEM((_CK, _D), jnp.float32),
        pltpu.VMEM_SHARED((_NP, _D), jnp.float32),
        pltpu.SemaphoreType.DMA,
        pltpu.SemaphoreType.DMA,
        pltpu.SemaphoreType.DMA,
        pltpu.SemaphoreType.DMA,
        pltpu.SemaphoreType.DMA,
        pltpu.SemaphoreType.DMA,
        pltpu.SemaphoreType.DMA,
        pltpu.SemaphoreType.DMA,
        pltpu.SemaphoreType.DMA,
        pltpu.SemaphoreType.DMA,
        pltpu.SemaphoreType.DMA,
        pltpu.SemaphoreType.DMA,
    ],
)
def _edge_pass(g_hbm, edge3_hbm, edge4_hbm, out0_hbm, out1_hbm, didx, sidx,
               rows0, rows1, acc, gsem0, gsem1, ssem0, ssem1, isem0, isem1,
               isem2, isem3, dsem0, dsem1, dsem2, dsem3):
    c = lax.axis_index("c")
    s = lax.axis_index("s")
    w = s * _NC + c
    rows = (rows0, rows1)
    gsem = (gsem0, gsem1)
    ssem = (ssem0, ssem1)
    isem = (isem0, isem1, isem2, isem3)
    dsem = (dsem0, dsem1, dsem2, dsem3)
    start_chunk = w * _CPW
    nchunks = _my_nchunks(w)

    zero = jnp.zeros((16,), jnp.float32)

    def zbody(i, carry):
        for j in range(_D // 16):
            rows0[i, pl.ds(j * 16, 16)] = zero
        return carry

    lax.fori_loop(0, _CK, zbody, 0)

    base_row = s * _RPS
    for j in range(_RPS // _CK):
        pltpu.sync_copy(rows0, acc.at[pl.ds(base_row + j * _CK, _CK)])
    plsc.subcore_barrier()

    def _idx_load(j, slot):
        return pltpu.make_async_copy(
            edge3_hbm.at[0, pl.ds(start_chunk + j, 1)],
            sidx.at[pl.ds(slot, 1)], isem[slot])

    def _didx_load(j, slot):
        return pltpu.make_async_copy(
            edge4_hbm.at[1, pl.ds(start_chunk + j, 1)],
            didx.at[pl.ds(slot, 1)], dsem[slot])

    def _gather(j, b, slot):
        return pltpu.make_async_copy(g_hbm.at[sidx.at[slot]], rows[b],
                                     gsem[b])

    def _scatter_half(b, slot, h):
        return pltpu.make_async_copy(
            rows[b].at[pl.ds(h * (_CK // 2), _CK // 2)],
            acc.at[didx.at[slot, h]], ssem[b])

    for k in range(4):
        @pl.when(jnp.int32(k) < nchunks)
        def _(k=k):
            _idx_load(k, k).start()
            _didx_load(k, k).start()

    for k in range(2):
        @pl.when(jnp.int32(k) < nchunks)
        def _(k=k):
            _idx_load(k, k).wait()
            _gather(k, k, k).start()

    def body(i, carry):
        for k in range(4):
            j = i * 4 + k
            b = k % 2

            @pl.when(j < nchunks)
            def _(j=j, b=b, k=k):
                _gather(j, b, k).wait()
                _didx_load(j, k).wait()
                _scatter_half(b, k, 0).start(add=True)
                _scatter_half(b, k, 1).start(add=True)
                _scatter_half(b, k, 0).wait()
                _scatter_half(b, k, 1).wait()

                @pl.when(j + 4 < nchunks)
                def _():
                    _idx_load(j + 4, k).start()
                    _didx_load(j + 4, k).start()

                @pl.when(j + 2 < nchunks)
                def _():
                    slot2 = (k + 2) % 4
                    _idx_load(j + 2, slot2).wait()
                    _gather(j + 2, b, slot2).start()

        return carry

    lax.fori_loop(0, _CPW // 4, body, 0)
    plsc.subcore_barrier()

    @pl.when(c == 0)
    def _():
        pltpu.sync_copy(acc.at[pl.ds(base_row, _RPS)],
                        out0_hbm.at[pl.ds(base_row, _RPS)])

    @pl.when(c == 1)
    def _():
        pltpu.sync_copy(acc.at[pl.ds(base_row, _RPS)],
                        out1_hbm.at[pl.ds(base_row, _RPS)])


def _layer_norm_block(t, g, b):
    mu = jnp.mean(t, axis=-1, keepdims=True)
    var = jnp.mean((t - mu) ** 2, axis=-1, keepdims=True)
    return (t - mu) * lax.rsqrt(var + 1e-5) * g + b


def _dinv_block(dega, degb):
    deg = dega[:, :1] + degb[:, :1] + 1.0
    return lax.rsqrt(deg)


def _emb_body(x_ref, we_ref, be_ref, ge_ref, bee_ref, wc_ref, h_ref,
              hw_ref):
    t = jnp.dot(x_ref[...], we_ref[...], precision=_HIGH,
                preferred_element_type=jnp.float32) + be_ref[...]
    h = jnp.maximum(_layer_norm_block(t, ge_ref[...], bee_ref[...]), 0.0)
    h_ref[...] = h
    hw_ref[...] = jnp.dot(h, wc_ref[...], precision=_HIGH,
                          preferred_element_type=jnp.float32)


def _scale_body(hw_ref, dega_ref, degb_ref, g_ref):
    dinv = _dinv_block(dega_ref[...], degb_ref[...])
    g_ref[...] = dinv * hw_ref[...]


def _post_body(sa_ref, sb_ref, g_ref, hp_ref, dega_ref, degb_ref, b_ref,
               gn_ref, bn_ref, wc_ref, h_ref, gout_ref):
    dinv = _dinv_block(dega_ref[...], degb_ref[...])
    conv = dinv * (sa_ref[...] + sb_ref[...] + g_ref[...]) + b_ref[...]
    h = jnp.maximum(_layer_norm_block(conv, gn_ref[...], bn_ref[...]), 0.0)
    h = h + hp_ref[...]
    h_ref[...] = h
    gout_ref[...] = dinv * jnp.dot(h, wc_ref[...], precision=_HIGH,
                                   preferred_element_type=jnp.float32)


def _final_body(sa_ref, sb_ref, g_ref, hp_ref, dega_ref, degb_ref, b_ref,
                gn_ref, bn_ref, batch_ref, wr1_ref, br1_ref, wr2_ref, br2_ref,
                out_ref, sums_ref, cnt_ref):
    i = pl.program_id(0)
    dinv = _dinv_block(dega_ref[...], degb_ref[...])
    conv = dinv * (sa_ref[...] + sb_ref[...] + g_ref[...]) + b_ref[...]
    h = jnp.maximum(_layer_norm_block(conv, gn_ref[...], bn_ref[...]), 0.0)
    h = h + hp_ref[...]

    row = batch_ref[0]
    seg = lax.broadcasted_iota(jnp.int32, (_G, _B), 0)
    onehot = (row == seg).astype(jnp.float32)
    psum = jnp.dot(onehot, h, precision=_HIGH,
                   preferred_element_type=jnp.float32)
    pcnt = jnp.dot(onehot, jnp.ones((_B, 8), jnp.float32), precision=_HIGH,
                   preferred_element_type=jnp.float32)

    @pl.when(i == 0)
    def _():
        sums_ref[...] = psum
        cnt_ref[...] = pcnt

    @pl.when(i > 0)
    def _():
        sums_ref[...] += psum
        cnt_ref[...] += pcnt

    @pl.when(i == _GRID - 1)
    def _():
        pooled = sums_ref[...] / jnp.maximum(cnt_ref[:, :1], 1.0)
        r = jnp.maximum(jnp.dot(pooled, wr1_ref[...], precision=_HIGH,
                                preferred_element_type=jnp.float32)
                        + br1_ref[...], 0.0)
        out_ref[...] = jnp.dot(r, wr2_ref[...], precision=_HIGH,
                               preferred_element_type=jnp.float32) + br2_ref[...]


def _row_spec():
    return pl.BlockSpec((_B, _D), lambda i: (i, 0))


def _deg_spec():
    return pl.BlockSpec((_B, _DW), lambda i: (i, 0))


def _full_spec(shape):
    return pl.BlockSpec(shape, lambda i: (0,) * len(shape))


def _emb_call(x, W_emb, b_emb, g_emb, be_emb, Wc0):
    return pl.pallas_call(
        _emb_body,
        grid=(_GRID,),
        in_specs=[
            _row_spec(),
            _full_spec((_D, _D)),
            _full_spec((1, _D)),
            _full_spec((1, _D)),
            _full_spec((1, _D)),
            _full_spec((_D, _D)),
        ],
        out_specs=[_row_spec(), _row_spec()],
        out_shape=[jax.ShapeDtypeStruct((_N, _D), jnp.float32)] * 2,
    )(x, W_emb, b_emb.reshape(1, _D), g_emb.reshape(1, _D),
      be_emb.reshape(1, _D), Wc0)


def _scale_call(hw, dega, degb):
    return pl.pallas_call(
        _scale_body,
        grid=(_GRID,),
        in_specs=[_row_spec(), _deg_spec(), _deg_spec()],
        out_specs=_row_spec(),
        out_shape=jax.ShapeDtypeStruct((_N, _D), jnp.float32),
    )(hw, dega, degb)


def _post_call(Sa, Sb, g, h, dega, degb, b, gn, bn, Wc_next):
    return pl.pallas_call(
        _post_body,
        grid=(_GRID,),
        in_specs=[
            _row_spec(),
            _row_spec(),
            _row_spec(),
            _row_spec(),
            _deg_spec(),
            _deg_spec(),
            _full_spec((1, _D)),
            _full_spec((1, _D)),
            _full_spec((1, _D)),
            _full_spec((_D, _D)),
        ],
        out_specs=[_row_spec(), _row_spec()],
        out_shape=[jax.ShapeDtypeStruct((_N, _D), jnp.float32)] * 2,
    )(Sa, Sb, g, h, dega, degb, b.reshape(1, _D), gn.reshape(1, _D),
      bn.reshape(1, _D), Wc_next)


def _final_call(Sa, Sb, g, h, dega, degb, b, gn, bn, batch3, W_r1, b_r1,
                W_r2, b_r2):
    return pl.pallas_call(
        _final_body,
        grid=(_GRID,),
        in_specs=[
            _row_spec(),
            _row_spec(),
            _row_spec(),
            _row_spec(),
            _deg_spec(),
            _deg_spec(),
            _full_spec((1, _D)),
            _full_spec((1, _D)),
            _full_spec((1, _D)),
            pl.BlockSpec((1, 1, _B), lambda i: (i, 0, 0)),
            _full_spec((_D, _G)),
            _full_spec((1, _G)),
            _full_spec((_G, _OUT)),
            _full_spec((1, _OUT)),
        ],
        out_specs=pl.BlockSpec((_G, _OUT), lambda i: (0, 0)),
        out_shape=jax.ShapeDtypeStruct((_G, _OUT), jnp.float32),
        scratch_shapes=[
            pltpu.VMEM((_G, _D), jnp.float32),
            pltpu.VMEM((_G, 8), jnp.float32),
        ],
    )(Sa, Sb, g, h, dega, degb, b.reshape(1, _D), gn.reshape(1, _D),
      bn.reshape(1, _D), batch3, W_r1, b_r1.reshape(1, _G),
      W_r2, b_r2.reshape(1, _OUT))


def kernel(x, edge_index, batch, W_emb, b_emb, g_emb, be_emb, Wc0, bc0, gn0,
           bn0, Wc1, bc1, gn1, bn1, Wc2, bc2, gn2, bn2, W_r1, b_r1, W_r2,
           b_r2):
    edge3 = edge_index.reshape(2, _NCHUNK, _CK)
    edge4 = edge_index.reshape(2, _NCHUNK, 2, _CK // 2)
    dega, degb = _deg_pass(edge3)

    h0, hw0 = _emb_call(x, W_emb, b_emb, g_emb, be_emb, Wc0)
    g0 = _scale_call(hw0, dega, degb)
    S0a, S0b = _edge_pass(g0, edge3, edge4)
    h1, g1 = _post_call(S0a, S0b, g0, h0, dega, degb, bc0, gn0, bn0, Wc1)
    S1a, S1b = _edge_pass(g1, edge3, edge4)
    h2, g2 = _post_call(S1a, S1b, g1, h1, dega, degb, bc1, gn1, bn1, Wc2)
    S2a, S2b = _edge_pass(g2, edge3, edge4)

    batch3 = batch.reshape(_GRID, 1, _B)
    return _final_call(S2a, S2b, g2, h2, dega, degb, bc2, gn2, bn2, batch3,
                       W_r1, b_r1, W_r2, b_r2)


# speedup vs baseline: 28.9752x; 1.0045x over previous
import functools

import jax
import jax.numpy as jnp
from jax import lax
from jax.experimental import pallas as pl
from jax.experimental.pallas import tpu as pltpu
from jax.experimental.pallas import tpu_sc as plsc

_N = 10000
_NP = 10240
_E = 320000
_D = 128
_G = 64
_OUT = 12

_NC = 2
_NS = 16
_NW = _NC * _NS
_CK = 128
_NCHUNK = _E // _CK
_CPW = 80
_CPW_LAST = _NCHUNK - _CPW * (_NW - 1)
_RPS = _NP // _NS
_DW = 16

_B = 2000
_GRID = _N // _B

_HIGH = jax.lax.Precision.HIGHEST


def _mesh():
    return plsc.VectorSubcoreMesh(core_axis_name="c", subcore_axis_name="s")


def _load_my_chunks(src2_hbm, buf, w):
    @pl.when(w < _NW - 1)
    def _():
        pltpu.sync_copy(src2_hbm.at[pl.ds(w * _CPW, _CPW)], buf)

    @pl.when(w == _NW - 1)
    def _():
        pltpu.sync_copy(src2_hbm.at[pl.ds((_NW - 1) * _CPW, _CPW_LAST)],
                        buf.at[pl.ds(0, _CPW_LAST)])


def _my_nchunks(w):
    return jnp.where(w < _NW - 1, _CPW, _CPW_LAST)


@functools.partial(
    pl.kernel,
    mesh=_mesh(),
    out_type=[jax.ShapeDtypeStruct((_NP, _DW), jnp.float32)] * 2,
    scratch_types=[
        pltpu.VMEM((_CPW, _CK), jnp.int32),
        pltpu.VMEM((_CK, _DW), jnp.float32),
        pltpu.VMEM_SHARED((_NP, _DW), jnp.float32),
    ],
)
def _deg_pass(edge3_hbm, out0_hbm, out1_hbm, didx_all, fill, acc):
    c = lax.axis_index("c")
    s = lax.axis_index("s")
    w = s * _NC + c

    def _fill(val):
        v = jnp.full((16,), val, jnp.float32)

        def body(i, carry):
            fill[i, pl.ds(0, 16)] = v
            return carry

        lax.fori_loop(0, _CK, body, 0)

    _fill(0.0)
    base_row = s * _RPS
    for j in range(_RPS // _CK):
        pltpu.sync_copy(fill, acc.at[pl.ds(base_row + j * _CK, _CK)])
    @pl.when(w < _NW - 1)
    def _():
        pltpu.sync_copy(edge3_hbm.at[1, pl.ds(w * _CPW, _CPW)], didx_all)

    @pl.when(w == _NW - 1)
    def _():
        pltpu.sync_copy(edge3_hbm.at[1, pl.ds((_NW - 1) * _CPW, _CPW_LAST)],
                        didx_all.at[pl.ds(0, _CPW_LAST)])

    nchunks = _my_nchunks(w)
    plsc.subcore_barrier()

    _fill(1.0)

    def body(j, carry):
        @pl.when(j < nchunks)
        def _():
            pltpu.sync_copy(fill, acc.at[didx_all.at[j]], add=True)

        return carry

    lax.fori_loop(0, _CPW, body, 0)
    plsc.subcore_barrier()

    @pl.when(c == 0)
    def _():
        pltpu.sync_copy(acc.at[pl.ds(base_row, _RPS)],
                        out0_hbm.at[pl.ds(base_row, _RPS)])

    @pl.when(c == 1)
    def _():
        pltpu.sync_copy(acc.at[pl.ds(base_row, _RPS)],
                        out1_hbm.at[pl.ds(base_row, _RPS)])


@functools.partial(
    pl.kernel,
    mesh=_mesh(),
    out_type=[jax.ShapeDtypeStruct((_NP, _D), jnp.float32)] * 2,
    scratch_types=[
        pltpu.VMEM((4, _CK), jnp.int32),
        pltpu.VMEM((4, _CK), jnp.int32),
        pltpu.VMEM((_CK, _D), jnp.float32),
        pltpu.VMEM((_CK, _D), jnp.float32),
        pltpu.VMEM_SHARED((_NP, _D), jnp.float32),
        pltpu.SemaphoreType.DMA,
        pltpu.SemaphoreType.DMA,
        pltpu.SemaphoreType.DMA,
        pltpu.SemaphoreType.DMA,
        pltpu.SemaphoreType.DMA,
        pltpu.SemaphoreType.DMA,
        pltpu.SemaphoreType.DMA,
        pltpu.SemaphoreType.DMA,
        pltpu.SemaphoreType.DMA,
        pltpu.SemaphoreType.DMA,
        pltpu.SemaphoreType.DMA,
        pltpu.SemaphoreType.DMA,
    ],
)
def _edge_pass(g_hbm, edge3_hbm, out0_hbm, out1_hbm, didx, sidx,
               rows0, rows1, acc, gsem0, gsem1, ssem0, ssem1, isem0, isem1,
               isem2, isem3, dsem0, dsem1, dsem2, dsem3):
    c = lax.axis_index("c")
    s = lax.axis_index("s")
    w = s * _NC + c
    rows = (rows0, rows1)
    gsem = (gsem0, gsem1)
    ssem = (ssem0, ssem1)
    isem = (isem0, isem1, isem2, isem3)
    dsem = (dsem0, dsem1, dsem2, dsem3)
    start_chunk = w * _CPW
    nchunks = _my_nchunks(w)

    zero = jnp.zeros((16,), jnp.float32)

    def zbody(i, carry):
        for j in range(_D // 16):
            rows0[i, pl.ds(j * 16, 16)] = zero
        return carry

    lax.fori_loop(0, _CK, zbody, 0)

    base_row = s * _RPS
    for j in range(_RPS // _CK):
        pltpu.sync_copy(rows0, acc.at[pl.ds(base_row + j * _CK, _CK)])
    plsc.subcore_barrier()

    def _idx_load(j, slot):
        return pltpu.make_async_copy(
            edge3_hbm.at[0, pl.ds(start_chunk + j, 1)],
            sidx.at[pl.ds(slot, 1)], isem[slot])

    def _didx_load(j, slot):
        return pltpu.make_async_copy(
            edge3_hbm.at[1, pl.ds(start_chunk + j, 1)],
            didx.at[pl.ds(slot, 1)], dsem[slot])

    def _gather(j, b, slot):
        return pltpu.make_async_copy(g_hbm.at[sidx.at[slot]], rows[b],
                                     gsem[b])

    def _scatter(b, slot):
        return pltpu.make_async_copy(rows[b], acc.at[didx.at[slot]], ssem[b])

    for k in range(4):
        @pl.when(jnp.int32(k) < nchunks)
        def _(k=k):
            _idx_load(k, k).start()
            _didx_load(k, k).start()

    for k in range(2):
        @pl.when(jnp.int32(k) < nchunks)
        def _(k=k):
            _idx_load(k, k).wait()
            _gather(k, k, k).start()

    def body(i, carry):
        for k in range(4):
            j = i * 4 + k
            b = k % 2

            @pl.when(j < nchunks)
            def _(j=j, b=b, k=k):
                _gather(j, b, k).wait()
                _didx_load(j, k).wait()
                _scatter(b, k).start(add=True)
                _scatter(b, k).wait()

                @pl.when(j + 4 < nchunks)
                def _():
                    _idx_load(j + 4, k).start()
                    _didx_load(j + 4, k).start()

                @pl.when(j + 2 < nchunks)
                def _():
                    slot2 = (k + 2) % 4
                    _idx_load(j + 2, slot2).wait()
                    _gather(j + 2, b, slot2).start()

        return carry

    lax.fori_loop(0, _CPW // 4, body, 0)
    plsc.subcore_barrier()

    @pl.when(c == 0)
    def _():
        pltpu.sync_copy(acc.at[pl.ds(base_row, _RPS)],
                        out0_hbm.at[pl.ds(base_row, _RPS)])

    @pl.when(c == 1)
    def _():
        pltpu.sync_copy(acc.at[pl.ds(base_row, _RPS)],
                        out1_hbm.at[pl.ds(base_row, _RPS)])


def _layer_norm_block(t, g, b):
    mu = jnp.mean(t, axis=-1, keepdims=True)
    var = jnp.mean((t - mu) ** 2, axis=-1, keepdims=True)
    return (t - mu) / jnp.sqrt(var + 1e-5) * g + b


def _dinv_block(dega, degb):
    deg = dega[:, :1] + degb[:, :1] + 1.0
    return 1.0 / jnp.sqrt(deg)


def _emb_body(x_ref, we_ref, be_ref, ge_ref, bee_ref, wc_ref, h_ref,
              hw_ref):
    t = jnp.dot(x_ref[...], we_ref[...], precision=_HIGH,
                preferred_element_type=jnp.float32) + be_ref[...]
    h = jnp.maximum(_layer_norm_block(t, ge_ref[...], bee_ref[...]), 0.0)
    h_ref[...] = h
    hw_ref[...] = jnp.dot(h, wc_ref[...], precision=_HIGH,
                          preferred_element_type=jnp.float32)


def _scale_body(hw_ref, dega_ref, degb_ref, g_ref):
    dinv = _dinv_block(dega_ref[...], degb_ref[...])
    g_ref[...] = dinv * hw_ref[...]


def _post_body(sa_ref, sb_ref, g_ref, hp_ref, dega_ref, degb_ref, b_ref,
               gn_ref, bn_ref, wc_ref, h_ref, gout_ref):
    dinv = _dinv_block(dega_ref[...], degb_ref[...])
    conv = dinv * (sa_ref[...] + sb_ref[...] + g_ref[...]) + b_ref[...]
    h = jnp.maximum(_layer_norm_block(conv, gn_ref[...], bn_ref[...]), 0.0)
    h = h + hp_ref[...]
    h_ref[...] = h
    gout_ref[...] = dinv * jnp.dot(h, wc_ref[...], precision=_HIGH,
                                   preferred_element_type=jnp.float32)


def _final_body(sa_ref, sb_ref, g_ref, hp_ref, dega_ref, degb_ref, b_ref,
                gn_ref, bn_ref, batch_ref, wr1_ref, br1_ref, wr2_ref, br2_ref,
                out_ref, sums_ref, cnt_ref):
    i = pl.program_id(0)
    dinv = _dinv_block(dega_ref[...], degb_ref[...])
    conv = dinv * (sa_ref[...] + sb_ref[...] + g_ref[...]) + b_ref[...]
    h = jnp.maximum(_layer_norm_block(conv, gn_ref[...], bn_ref[...]), 0.0)
    h = h + hp_ref[...]

    row = batch_ref[0]
    seg = lax.broadcasted_iota(jnp.int32, (_G, _B), 0)
    onehot = (row == seg).astype(jnp.float32)
    psum = jnp.dot(onehot, h, precision=_HIGH,
                   preferred_element_type=jnp.float32)
    pcnt = jnp.dot(onehot, jnp.ones((_B, 8), jnp.float32), precision=_HIGH,
                   preferred_element_type=jnp.float32)

    @pl.when(i == 0)
    def _():
        sums_ref[...] = psum
        cnt_ref[...] = pcnt

    @pl.when(i > 0)
    def _():
        sums_ref[...] += psum
        cnt_ref[...] += pcnt

    @pl.when(i == _GRID - 1)
    def _():
        pooled = sums_ref[...] / jnp.maximum(cnt_ref[:, :1], 1.0)
        r = jnp.maximum(jnp.dot(pooled, wr1_ref[...], precision=_HIGH,
                                preferred_element_type=jnp.float32)
                        + br1_ref[...], 0.0)
        out_ref[...] = jnp.dot(r, wr2_ref[...], precision=_HIGH,
                               preferred_element_type=jnp.float32) + br2_ref[...]


def _row_spec():
    return pl.BlockSpec((_B, _D), lambda i: (i, 0))


def _deg_spec():
    return pl.BlockSpec((_B, _DW), lambda i: (i, 0))


def _full_spec(shape):
    return pl.BlockSpec(shape, lambda i: (0,) * len(shape))


def _emb_call(x, W_emb, b_emb, g_emb, be_emb, Wc0):
    return pl.pallas_call(
        _emb_body,
        grid=(_GRID,),
        in_specs=[
            _row_spec(),
            _full_spec((_D, _D)),
            _full_spec((1, _D)),
            _full_spec((1, _D)),
            _full_spec((1, _D)),
            _full_spec((_D, _D)),
        ],
        out_specs=[_row_spec(), _row_spec()],
        out_shape=[jax.ShapeDtypeStruct((_N, _D), jnp.float32)] * 2,
    )(x, W_emb, b_emb.reshape(1, _D), g_emb.reshape(1, _D),
      be_emb.reshape(1, _D), Wc0)


def _scale_call(hw, dega, degb):
    return pl.pallas_call(
        _scale_body,
        grid=(_GRID,),
        in_specs=[_row_spec(), _deg_spec(), _deg_spec()],
        out_specs=_row_spec(),
        out_shape=jax.ShapeDtypeStruct((_N, _D), jnp.float32),
    )(hw, dega, degb)


def _post_call(Sa, Sb, g, h, dega, degb, b, gn, bn, Wc_next):
    return pl.pallas_call(
        _post_body,
        grid=(_GRID,),
        in_specs=[
            _row_spec(),
            _row_spec(),
            _row_spec(),
            _row_spec(),
            _deg_spec(),
            _deg_spec(),
            _full_spec((1, _D)),
            _full_spec((1, _D)),
            _full_spec((1, _D)),
            _full_spec((_D, _D)),
        ],
        out_specs=[_row_spec(), _row_spec()],
        out_shape=[jax.ShapeDtypeStruct((_N, _D), jnp.float32)] * 2,
    )(Sa, Sb, g, h, dega, degb, b.reshape(1, _D), gn.reshape(1, _D),
      bn.reshape(1, _D), Wc_next)


def _final_call(Sa, Sb, g, h, dega, degb, b, gn, bn, batch3, W_r1, b_r1,
                W_r2, b_r2):
    return pl.pallas_call(
        _final_body,
        grid=(_GRID,),
        in_specs=[
            _row_spec(),
            _row_spec(),
            _row_spec(),
            _row_spec(),
            _deg_spec(),
            _deg_spec(),
            _full_spec((1, _D)),
            _full_spec((1, _D)),
            _full_spec((1, _D)),
            pl.BlockSpec((1, 1, _B), lambda i: (i, 0, 0)),
            _full_spec((_D, _G)),
            _full_spec((1, _G)),
            _full_spec((_G, _OUT)),
            _full_spec((1, _OUT)),
        ],
        out_specs=pl.BlockSpec((_G, _OUT), lambda i: (0, 0)),
        out_shape=jax.ShapeDtypeStruct((_G, _OUT), jnp.float32),
        scratch_shapes=[
            pltpu.VMEM((_G, _D), jnp.float32),
            pltpu.VMEM((_G, 8), jnp.float32),
        ],
    )(Sa, Sb, g, h, dega, degb, b.reshape(1, _D), gn.reshape(1, _D),
      bn.reshape(1, _D), batch3, W_r1, b_r1.reshape(1, _G),
      W_r2, b_r2.reshape(1, _OUT))


def kernel(x, edge_index, batch, W_emb, b_emb, g_emb, be_emb, Wc0, bc0, gn0,
           bn0, Wc1, bc1, gn1, bn1, Wc2, bc2, gn2, bn2, W_r1, b_r1, W_r2,
           b_r2):
    edge3 = edge_index.reshape(2, _NCHUNK, _CK)
    dega, degb = _deg_pass(edge3)

    h0, hw0 = _emb_call(x, W_emb, b_emb, g_emb, be_emb, Wc0)
    g0 = _scale_call(hw0, dega, degb)
    S0a, S0b = _edge_pass(g0, edge3)
    h1, g1 = _post_call(S0a, S0b, g0, h0, dega, degb, bc0, gn0, bn0, Wc1)
    S1a, S1b = _edge_pass(g1, edge3)
    h2, g2 = _post_call(S1a, S1b, g1, h1, dega, degb, bc1, gn1, bn1, Wc2)
    S2a, S2b = _edge_pass(g2, edge3)

    batch3 = batch.reshape(_GRID, 1, _B)
    return _final_call(S2a, S2b, g2, h2, dega, degb, bc2, gn2, bn2, batch3,
                       W_r1, b_r1, W_r2, b_r2)


# speedup vs baseline: 29.0191x; 1.0015x over previous
import functools

import jax
import jax.numpy as jnp
from jax import lax
from jax.experimental import pallas as pl
from jax.experimental.pallas import tpu as pltpu
from jax.experimental.pallas import tpu_sc as plsc

_N = 10000
_NP = 10240
_E = 320000
_D = 128
_G = 64
_OUT = 12

_NC = 2
_NS = 16
_NW = _NC * _NS
_CK = 128
_NCHUNK = _E // _CK
_CPW = 80
_CPW_LAST = _NCHUNK - _CPW * (_NW - 1)
_RPS = _NP // _NS
_DW = 16

_B = 2000
_GRID = _N // _B

_HIGH = jax.lax.Precision.HIGHEST


def _mesh():
    return plsc.VectorSubcoreMesh(core_axis_name="c", subcore_axis_name="s")


def _my_nchunks(w):
    return jnp.where(w < _NW - 1, _CPW, _CPW_LAST)


@functools.partial(
    pl.kernel,
    mesh=_mesh(),
    out_type=[jax.ShapeDtypeStruct((_NP, _DW), jnp.float32)] * 2,
    scratch_types=[
        pltpu.VMEM((_CPW, _CK), jnp.int32),
        pltpu.VMEM((_CK, _DW), jnp.float32),
        pltpu.VMEM_SHARED((_NP, _DW), jnp.float32),
    ],
)
def _deg_pass(edge3_hbm, out0_hbm, out1_hbm, didx_all, fill, acc):
    c = lax.axis_index("c")
    s = lax.axis_index("s")
    w = s * _NC + c

    def _fill(val):
        v = jnp.full((16,), val, jnp.float32)

        def body(i, carry):
            fill[i, pl.ds(0, 16)] = v
            return carry

        lax.fori_loop(0, _CK, body, 0)

    _fill(0.0)
    base_row = s * _RPS
    for j in range(_RPS // _CK):
        pltpu.sync_copy(fill, acc.at[pl.ds(base_row + j * _CK, _CK)])
    @pl.when(w < _NW - 1)
    def _():
        pltpu.sync_copy(edge3_hbm.at[1, pl.ds(w * _CPW, _CPW)], didx_all)

    @pl.when(w == _NW - 1)
    def _():
        pltpu.sync_copy(edge3_hbm.at[1, pl.ds((_NW - 1) * _CPW, _CPW_LAST)],
                        didx_all.at[pl.ds(0, _CPW_LAST)])

    nchunks = _my_nchunks(w)
    plsc.subcore_barrier()

    _fill(1.0)

    def body(j, carry):
        @pl.when(j < nchunks)
        def _():
            pltpu.sync_copy(fill, acc.at[didx_all.at[j]], add=True)

        return carry

    lax.fori_loop(0, _CPW, body, 0)
    plsc.subcore_barrier()

    @pl.when(c == 0)
    def _():
        pltpu.sync_copy(acc.at[pl.ds(base_row, _RPS)],
                        out0_hbm.at[pl.ds(base_row, _RPS)])

    @pl.when(c == 1)
    def _():
        pltpu.sync_copy(acc.at[pl.ds(base_row, _RPS)],
                        out1_hbm.at[pl.ds(base_row, _RPS)])


@functools.partial(
    pl.kernel,
    mesh=_mesh(),
    out_type=[jax.ShapeDtypeStruct((_NP, _D), jnp.float32)] * 2,
    scratch_types=[
        pltpu.VMEM((4, _CK), jnp.int32),
        pltpu.VMEM((4, _CK), jnp.int32),
        pltpu.VMEM((_CK, _D), jnp.float32),
        pltpu.VMEM((_CK, _D), jnp.float32),
        pltpu.VMEM_SHARED((_NP, _D), jnp.float32),
        pltpu.SemaphoreType.DMA,
        pltpu.SemaphoreType.DMA,
        pltpu.SemaphoreType.DMA,
        pltpu.SemaphoreType.DMA,
        pltpu.SemaphoreType.DMA,
        pltpu.SemaphoreType.DMA,
        pltpu.SemaphoreType.DMA,
        pltpu.SemaphoreType.DMA,
        pltpu.SemaphoreType.DMA,
        pltpu.SemaphoreType.DMA,
        pltpu.SemaphoreType.DMA,
        pltpu.SemaphoreType.DMA,
    ],
)
def _edge_pass(g_hbm, edge3_hbm, out0_hbm, out1_hbm, didx, sidx,
               rows0, rows1, acc, gsem0, gsem1, ssem0, ssem1, isem0, isem1,
               isem2, isem3, dsem0, dsem1, dsem2, dsem3):
    c = lax.axis_index("c")
    s = lax.axis_index("s")
    w = s * _NC + c
    rows = (rows0, rows1)
    gsem = (gsem0, gsem1)
    ssem = (ssem0, ssem1)
    isem = (isem0, isem1, isem2, isem3)
    dsem = (dsem0, dsem1, dsem2, dsem3)
    start_chunk = w * _CPW
    nchunks = _my_nchunks(w)

    zero = jnp.zeros((16,), jnp.float32)

    def zbody(i, carry):
        for j in range(_D // 16):
            rows0[i, pl.ds(j * 16, 16)] = zero
        return carry

    lax.fori_loop(0, _CK, zbody, 0)

    base_row = s * _RPS
    for j in range(_RPS // _CK):
        pltpu.sync_copy(rows0, acc.at[pl.ds(base_row + j * _CK, _CK)])
    plsc.subcore_barrier()

    def _idx_load(j, slot):
        return pltpu.make_async_copy(
            edge3_hbm.at[0, pl.ds(start_chunk + j, 1)],
            sidx.at[pl.ds(slot, 1)], isem[slot])

    def _didx_load(j, slot):
        return pltpu.make_async_copy(
            edge3_hbm.at[1, pl.ds(start_chunk + j, 1)],
            didx.at[pl.ds(slot, 1)], dsem[slot])

    def _gather(j, b, slot):
        return pltpu.make_async_copy(g_hbm.at[sidx.at[slot]], rows[b],
                                     gsem[b])

    def _scatter(b, slot):
        return pltpu.make_async_copy(rows[b], acc.at[didx.at[slot]], ssem[b])

    for k in range(4):
        @pl.when(jnp.int32(k) < nchunks)
        def _(k=k):
            _idx_load(k, k).start()
            _didx_load(k, k).start()

    for k in range(2):
        @pl.when(jnp.int32(k) < nchunks)
        def _(k=k):
            _idx_load(k, k).wait()
            _gather(k, k, k).start()

    def body(i, carry):
        for k in range(4):
            j = i * 4 + k
            b = k % 2

            @pl.when(j < nchunks)
            def _(j=j, b=b, k=k):
                _gather(j, b, k).wait()
                _didx_load(j, k).wait()
                _scatter(b, k).start(add=True)
                _scatter(b, k).wait()

                @pl.when(j + 4 < nchunks)
                def _():
                    _idx_load(j + 4, k).start()
                    _didx_load(j + 4, k).start()

                @pl.when(j + 2 < nchunks)
                def _():
                    slot2 = (k + 2) % 4
                    _idx_load(j + 2, slot2).wait()
                    _gather(j + 2, b, slot2).start()

        return carry

    lax.fori_loop(0, _CPW // 4, body, 0)
    plsc.subcore_barrier()

    @pl.when(c == 0)
    def _():
        pltpu.sync_copy(acc.at[pl.ds(base_row, _RPS)],
                        out0_hbm.at[pl.ds(base_row, _RPS)])

    @pl.when(c == 1)
    def _():
        pltpu.sync_copy(acc.at[pl.ds(base_row, _RPS)],
                        out1_hbm.at[pl.ds(base_row, _RPS)])


def _layer_norm_block(t, g, b):
    mu = jnp.mean(t, axis=-1, keepdims=True)
    var = jnp.mean((t - mu) ** 2, axis=-1, keepdims=True)
    return (t - mu) / jnp.sqrt(var + 1e-5) * g + b


def _dinv_block(dega, degb):
    deg = dega[:, :1] + degb[:, :1] + 1.0
    return 1.0 / jnp.sqrt(deg)


def _emb_body(x_ref, we_ref, be_ref, ge_ref, bee_ref, wc_ref, h_ref,
              hw_ref):
    t = jnp.dot(x_ref[...], we_ref[...], precision=_HIGH,
                preferred_element_type=jnp.float32) + be_ref[...]
    h = jnp.maximum(_layer_norm_block(t, ge_ref[...], bee_ref[...]), 0.0)
    h_ref[...] = h
    hw_ref[...] = jnp.dot(h, wc_ref[...], precision=_HIGH,
                          preferred_element_type=jnp.float32)


def _scale_body(hw_ref, dega_ref, degb_ref, g_ref):
    dinv = _dinv_block(dega_ref[...], degb_ref[...])
    g_ref[...] = dinv * hw_ref[...]


def _post_body(sa_ref, sb_ref, g_ref, hp_ref, dega_ref, degb_ref, b_ref,
               gn_ref, bn_ref, wc_ref, h_ref, gout_ref):
    dinv = _dinv_block(dega_ref[...], degb_ref[...])
    conv = dinv * (sa_ref[...] + sb_ref[...] + g_ref[...]) + b_ref[...]
    h = jnp.maximum(_layer_norm_block(conv, gn_ref[...], bn_ref[...]), 0.0)
    h = h + hp_ref[...]
    h_ref[...] = h
    gout_ref[...] = dinv * jnp.dot(h, wc_ref[...], precision=_HIGH,
                                   preferred_element_type=jnp.float32)


def _final_body(sa_ref, sb_ref, g_ref, hp_ref, dega_ref, degb_ref, b_ref,
                gn_ref, bn_ref, batch_ref, wr1_ref, br1_ref, wr2_ref, br2_ref,
                out_ref, sums_ref, cnt_ref):
    i = pl.program_id(0)
    dinv = _dinv_block(dega_ref[...], degb_ref[...])
    conv = dinv * (sa_ref[...] + sb_ref[...] + g_ref[...]) + b_ref[...]
    h = jnp.maximum(_layer_norm_block(conv, gn_ref[...], bn_ref[...]), 0.0)
    h = h + hp_ref[...]

    row = batch_ref[0]
    seg = lax.broadcasted_iota(jnp.int32, (_G, _B), 0)
    onehot = (row == seg).astype(jnp.float32)
    psum = jnp.dot(onehot, h, precision=_HIGH,
                   preferred_element_type=jnp.float32)
    pcnt = jnp.dot(onehot, jnp.ones((_B, 8), jnp.float32), precision=_HIGH,
                   preferred_element_type=jnp.float32)

    @pl.when(i == 0)
    def _():
        sums_ref[...] = psum
        cnt_ref[...] = pcnt

    @pl.when(i > 0)
    def _():
        sums_ref[...] += psum
        cnt_ref[...] += pcnt

    @pl.when(i == _GRID - 1)
    def _():
        pooled = sums_ref[...] / jnp.maximum(cnt_ref[:, :1], 1.0)
        r = jnp.maximum(jnp.dot(pooled, wr1_ref[...], precision=_HIGH,
                                preferred_element_type=jnp.float32)
                        + br1_ref[...], 0.0)
        out_ref[...] = jnp.dot(r, wr2_ref[...], precision=_HIGH,
                               preferred_element_type=jnp.float32) + br2_ref[...]


def _row_spec():
    return pl.BlockSpec((_B, _D), lambda i: (i, 0))


def _deg_spec():
    return pl.BlockSpec((_B, _DW), lambda i: (i, 0))


def _full_spec(shape):
    return pl.BlockSpec(shape, lambda i: (0,) * len(shape))


def _emb_call(x, W_emb, b_emb, g_emb, be_emb, Wc0):
    return pl.pallas_call(
        _emb_body,
        grid=(_GRID,),
        in_specs=[
            _row_spec(),
            _full_spec((_D, _D)),
            _full_spec((1, _D)),
            _full_spec((1, _D)),
            _full_spec((1, _D)),
            _full_spec((_D, _D)),
        ],
        out_specs=[_row_spec(), _row_spec()],
        out_shape=[jax.ShapeDtypeStruct((_N, _D), jnp.float32)] * 2,
    )(x, W_emb, b_emb.reshape(1, _D), g_emb.reshape(1, _D),
      be_emb.reshape(1, _D), Wc0)


def _scale_call(hw, dega, degb):
    return pl.pallas_call(
        _scale_body,
        grid=(_GRID,),
        in_specs=[_row_spec(), _deg_spec(), _deg_spec()],
        out_specs=_row_spec(),
        out_shape=jax.ShapeDtypeStruct((_N, _D), jnp.float32),
    )(hw, dega, degb)


def _post_call(Sa, Sb, g, h, dega, degb, b, gn, bn, Wc_next):
    return pl.pallas_call(
        _post_body,
        grid=(_GRID,),
        in_specs=[
            _row_spec(),
            _row_spec(),
            _row_spec(),
            _row_spec(),
            _deg_spec(),
            _deg_spec(),
            _full_spec((1, _D)),
            _full_spec((1, _D)),
            _full_spec((1, _D)),
            _full_spec((_D, _D)),
        ],
        out_specs=[_row_spec(), _row_spec()],
        out_shape=[jax.ShapeDtypeStruct((_N, _D), jnp.float32)] * 2,
    )(Sa, Sb, g, h, dega, degb, b.reshape(1, _D), gn.reshape(1, _D),
      bn.reshape(1, _D), Wc_next)


def _final_call(Sa, Sb, g, h, dega, degb, b, gn, bn, batch3, W_r1, b_r1,
                W_r2, b_r2):
    return pl.pallas_call(
        _final_body,
        grid=(_GRID,),
        in_specs=[
            _row_spec(),
            _row_spec(),
            _row_spec(),
            _row_spec(),
            _deg_spec(),
            _deg_spec(),
            _full_spec((1, _D)),
            _full_spec((1, _D)),
            _full_spec((1, _D)),
            pl.BlockSpec((1, 1, _B), lambda i: (i, 0, 0)),
            _full_spec((_D, _G)),
            _full_spec((1, _G)),
            _full_spec((_G, _OUT)),
            _full_spec((1, _OUT)),
        ],
        out_specs=pl.BlockSpec((_G, _OUT), lambda i: (0, 0)),
        out_shape=jax.ShapeDtypeStruct((_G, _OUT), jnp.float32),
        scratch_shapes=[
            pltpu.VMEM((_G, _D), jnp.float32),
            pltpu.VMEM((_G, 8), jnp.float32),
        ],
    )(Sa, Sb, g, h, dega, degb, b.reshape(1, _D), gn.reshape(1, _D),
      bn.reshape(1, _D), batch3, W_r1, b_r1.reshape(1, _G),
      W_r2, b_r2.reshape(1, _OUT))


def kernel(x, edge_index, batch, W_emb, b_emb, g_emb, be_emb, Wc0, bc0, gn0,
           bn0, Wc1, bc1, gn1, bn1, Wc2, bc2, gn2, bn2, W_r1, b_r1, W_r2,
           b_r2):
    edge3 = edge_index.reshape(2, _NCHUNK, _CK)
    dega, degb = _deg_pass(edge3)

    h0, hw0 = _emb_call(x, W_emb, b_emb, g_emb, be_emb, Wc0)
    g0 = _scale_call(hw0, dega, degb)
    S0a, S0b = _edge_pass(g0, edge3)
    h1, g1 = _post_call(S0a, S0b, g0, h0, dega, degb, bc0, gn0, bn0, Wc1)
    S1a, S1b = _edge_pass(g1, edge3)
    h2, g2 = _post_call(S1a, S1b, g1, h1, dega, degb, bc1, gn1, bn1, Wc2)
    S2a, S2b = _edge_pass(g2, edge3)

    batch3 = batch.reshape(_GRID, 1, _B)
    return _final_call(S2a, S2b, g2, h2, dega, degb, bc2, gn2, bn2, batch3,
                       W_r1, b_r1, W_r2, b_r2)
